# baseline (speedup 1.0000x reference)
import functools

import jax
import jax.numpy as jnp
from jax import lax
from jax.experimental import pallas as pl
from jax.experimental.pallas import tpu as pltpu
from jax.experimental.pallas import tpu_sc as plsc

IN_G = 32
OUT_G = 32
IN_MEM = 64
OUT_MEM = 64
N_EMBED = 1024
DIM = IN_MEM * OUT_MEM
NROWS = IN_G * OUT_G
NUMEL = NROWS * DIM


def _transpose_body(e_ref, o_ref):
    o_ref[...] = e_ref[...].T


def _transpose(embed):
    return pl.pallas_call(
        _transpose_body,
        grid=(8,),
        in_specs=[pl.BlockSpec((512, N_EMBED), lambda i: (i, 0))],
        out_specs=pl.BlockSpec((N_EMBED, 512), lambda i: (0, i)),
        out_shape=jax.ShapeDtypeStruct((N_EMBED, DIM), jnp.float32),
    )(embed)


def _dist_body(f_ref, e_ref, ie_ref, diff_ref):
    F = f_ref[...]
    E = e_ref[...]
    fe = jnp.dot(F, E, preferred_element_type=jnp.float32)
    enorm = jnp.sum(E * E, axis=0, keepdims=True)
    fnorm = jnp.sum(F * F, axis=1, keepdims=True)
    dist = fnorm - 2.0 * fe + enorm
    rowmin = jnp.min(dist, axis=1, keepdims=True)
    iota = lax.broadcasted_iota(jnp.int32, dist.shape, 1)
    big = jnp.int32(N_EMBED)
    idx = jnp.min(jnp.where(dist <= rowmin, iota, big), axis=1)
    ie_ref[0, 0, :] = idx.astype(jnp.int32)

    @pl.when(pl.program_id(0) == 0)
    def _():
        diff_ref[0, 0] = 0.0

    diff_ref[0, 0] += jnp.sum(rowmin) * (1.0 / NUMEL)


def _dist_argmin(flatten, embed):
    return pl.pallas_call(
        _dist_body,
        grid=(8,),
        in_specs=[
            pl.BlockSpec((128, DIM), lambda i: (i, 0)),
            pl.BlockSpec((DIM, N_EMBED), lambda i: (0, 0)),
        ],
        out_specs=[
            pl.BlockSpec((1, 1, 128), lambda i: (i, 0, 0)),
            pl.BlockSpec((1, 1), lambda i: (0, 0),
                         memory_space=pltpu.SMEM),
        ],
        out_shape=[
            jax.ShapeDtypeStruct((8, 1, 128), jnp.int32),
            jax.ShapeDtypeStruct((1, 1), jnp.float32),
        ],
    )(flatten, embed)


_SC_CHUNK = 8


def _sc_gather(embedT, idx):
    info = plsc.get_sparse_core_info()
    nc, ns = info.num_cores, info.num_subcores
    nw = nc * ns
    rows_per_w = NROWS // nw
    nchunks = rows_per_w // _SC_CHUNK
    mesh = plsc.VectorSubcoreMesh(core_axis_name="c", subcore_axis_name="s")

    @functools.partial(
        pl.kernel,
        mesh=mesh,
        out_type=jax.ShapeDtypeStruct((NROWS, DIM), jnp.float32),
        scratch_types=[
            pltpu.VMEM((rows_per_w,), jnp.int32),
            pltpu.VMEM((2, _SC_CHUNK, DIM), jnp.float32),
            pltpu.SemaphoreType.DMA,
            pltpu.SemaphoreType.DMA,
            pltpu.SemaphoreType.DMA,
        ],
    )
    def k(table_hbm, idx_hbm, q2_hbm, idx_v, bufs, sem_g, sem_f0, sem_f1):
        wid = lax.axis_index("s") * nc + lax.axis_index("c")
        base = wid * rows_per_w
        pltpu.sync_copy(idx_hbm.at[pl.ds(base, rows_per_w)], idx_v)
        fsems = (sem_f0, sem_f1)
        flushes = [None, None]
        for ch in range(nchunks):
            b = ch % 2
            if flushes[b] is not None:
                flushes[b].wait()
            pltpu.async_copy(
                table_hbm.at[idx_v.at[pl.ds(ch * _SC_CHUNK, _SC_CHUNK)]],
                bufs.at[b], sem_g).wait()
            flushes[b] = pltpu.async_copy(
                bufs.at[b],
                q2_hbm.at[pl.ds(base + ch * _SC_CHUNK, _SC_CHUNK)],
                fsems[b])
        for fl in flushes:
            fl.wait()

    return k(embedT, idx)


def _place_body(q_ref, o_ref):
    a = q_ref[0]
    o_ref[:, pl.ds(pl.program_id(1) * 128, 128)] = jnp.concatenate(
        [a[0], a[1]], axis=1)


def _place(q4):
    return pl.pallas_call(
        _place_body,
        grid=(IN_G, OUT_G // 2),
        in_specs=[pl.BlockSpec((1, 2, IN_MEM, OUT_MEM),
                               lambda g, p: (g, p, 0, 0))],
        out_specs=pl.BlockSpec((IN_MEM, OUT_G * OUT_MEM), lambda g, p: (g, 0)),
        out_shape=jax.ShapeDtypeStruct((IN_G * IN_MEM, OUT_G * OUT_MEM),
                                       jnp.float32),
    )(q4)


def _matmul_body(x_ref, w_ref, o_ref):
    o_ref[...] = jnp.dot(x_ref[...], w_ref[...],
                         preferred_element_type=jnp.float32)


def _matmul(x, qw):
    B = x.shape[0]
    K = x.shape[1]
    O = qw.shape[1]
    bB, bO = 1024, 512
    return pl.pallas_call(
        _matmul_body,
        grid=(B // bB, O // bO),
        in_specs=[
            pl.BlockSpec((bB, K), lambda i, j: (i, 0)),
            pl.BlockSpec((K, bO), lambda i, j: (0, j)),
        ],
        out_specs=pl.BlockSpec((bB, bO), lambda i, j: (i, j)),
        out_shape=jax.ShapeDtypeStruct((B, O), jnp.float32),
    )(x, qw)


def kernel(x, weight, embed, use_qw):
    flatten = weight.reshape(NROWS, DIM)
    embedT = _transpose(embed)
    idx8, diff11 = _dist_argmin(flatten, embed)
    q2 = _sc_gather(embedT, idx8.reshape(NROWS))
    qw = _place(q2.reshape(IN_G, OUT_G, IN_MEM, OUT_MEM))
    res = _matmul(x, qw)
    flag = jnp.asarray(use_qw) != 0
    diff = jnp.where(flag, diff11.reshape(()), jnp.float32(0.0))
    return (res, diff)


# speedup vs baseline: 2.2452x; 2.2452x over previous
import functools

import jax
import jax.numpy as jnp
from jax import lax
from jax.experimental import pallas as pl
from jax.experimental.pallas import tpu as pltpu
from jax.experimental.pallas import tpu_sc as plsc

IN_G = 32
OUT_G = 32
IN_MEM = 64
OUT_MEM = 64
N_EMBED = 1024
DIM = IN_MEM * OUT_MEM
NROWS = IN_G * OUT_G
NUMEL = NROWS * DIM


def _dist_body(f_ref, e_ref, ie_ref, diff_ref, et_ref):
    F = f_ref[...]
    E = e_ref[...]
    fe = jnp.dot(F, E, preferred_element_type=jnp.float32)
    enorm = jnp.sum(E * E, axis=0, keepdims=True)
    fnorm = jnp.sum(F * F, axis=1, keepdims=True)
    dist = fnorm - 2.0 * fe + enorm
    rowmin = jnp.min(dist, axis=1, keepdims=True)
    iota = lax.broadcasted_iota(jnp.int32, dist.shape, 1)
    big = jnp.int32(N_EMBED)
    idx = jnp.min(jnp.where(dist <= rowmin, iota, big), axis=1)
    ie_ref[0, 0, :] = idx.astype(jnp.int32)
    et_ref[...] = e_ref[pl.ds(pl.program_id(0) * 512, 512), :].T

    @pl.when(pl.program_id(0) == 0)
    def _():
        diff_ref[0, 0] = 0.0

    diff_ref[0, 0] += jnp.sum(rowmin) * (1.0 / NUMEL)


def _dist_argmin(flatten, embed):
    return pl.pallas_call(
        _dist_body,
        grid=(8,),
        in_specs=[
            pl.BlockSpec((128, DIM), lambda i: (i, 0)),
            pl.BlockSpec((DIM, N_EMBED), lambda i: (0, 0)),
        ],
        out_specs=[
            pl.BlockSpec((1, 1, 128), lambda i: (i, 0, 0)),
            pl.BlockSpec((1, 1), lambda i: (0, 0),
                         memory_space=pltpu.SMEM),
            pl.BlockSpec((N_EMBED, 512), lambda i: (0, i)),
        ],
        out_shape=[
            jax.ShapeDtypeStruct((8, 1, 128), jnp.int32),
            jax.ShapeDtypeStruct((1, 1), jnp.float32),
            jax.ShapeDtypeStruct((N_EMBED, DIM), jnp.float32),
        ],
    )(flatten, embed)


_SC_CHUNK = 8


def _sc_gather(embedT, idx):
    info = plsc.get_sparse_core_info()
    nc, ns = info.num_cores, info.num_subcores
    nw = nc * ns
    rows_per_w = NROWS // nw
    nchunks = rows_per_w // _SC_CHUNK
    mesh = plsc.VectorSubcoreMesh(core_axis_name="c", subcore_axis_name="s")

    @functools.partial(
        pl.kernel,
        mesh=mesh,
        out_type=jax.ShapeDtypeStruct((IN_G * IN_MEM, OUT_G * OUT_MEM),
                                      jnp.float32),
        scratch_types=[
            pltpu.VMEM((rows_per_w,), jnp.int32),
            pltpu.VMEM((2, _SC_CHUNK, DIM), jnp.float32),
            pltpu.VMEM((IN_MEM, _SC_CHUNK * OUT_MEM), jnp.float32),
            pltpu.SemaphoreType.DMA,
            pltpu.SemaphoreType.DMA,
        ],
    )
    def k(table_hbm, idx_hbm, qw_hbm, idx_v, rows_v, slab_v, sem_g, sem_f):
        wid = lax.axis_index("s") * nc + lax.axis_index("c")
        base = wid * rows_per_w
        slab_w = _SC_CHUNK * OUT_MEM
        pltpu.sync_copy(idx_hbm.at[pl.ds(base, rows_per_w)], idx_v)

        def gather(ch):
            return pltpu.async_copy(
                table_hbm.at[idx_v.at[pl.ds(ch * _SC_CHUNK, _SC_CHUNK)]],
                rows_v.at[ch % 2], sem_g)

        pending = gather(0)
        for ch in range(nchunks):
            b = ch % 2
            pending.wait()
            if ch + 1 < nchunks:
                pending = gather(ch + 1)

            def body(im, _):
                for j in range(_SC_CHUNK):
                    for s in range(OUT_MEM // 16):
                        slab_v[im, pl.ds(j * OUT_MEM + s * 16, 16)] = (
                            rows_v[b, j, pl.ds(im * OUT_MEM + s * 16, 16)])
                return _

            lax.fori_loop(0, IN_MEM, body, None, unroll=False)
            pltpu.sync_copy(
                slab_v,
                qw_hbm.at[pl.ds(wid * IN_MEM, IN_MEM),
                          pl.ds(ch * slab_w, slab_w)])

    return k(embedT, idx)


def _matmul_body(x_ref, w_ref, o_ref):
    o_ref[...] = jnp.dot(x_ref[...], w_ref[...],
                         preferred_element_type=jnp.float32)


def _matmul(x, qw):
    B = x.shape[0]
    K = x.shape[1]
    O = qw.shape[1]
    bB, bO = 1024, 512
    return pl.pallas_call(
        _matmul_body,
        grid=(B // bB, O // bO),
        in_specs=[
            pl.BlockSpec((bB, K), lambda i, j: (i, 0)),
            pl.BlockSpec((K, bO), lambda i, j: (0, j)),
        ],
        out_specs=pl.BlockSpec((bB, bO), lambda i, j: (i, j)),
        out_shape=jax.ShapeDtypeStruct((B, O), jnp.float32),
    )(x, qw)


def kernel(x, weight, embed, use_qw):
    flatten = weight.reshape(NROWS, DIM)
    idx8, diff11, embedT = _dist_argmin(flatten, embed)
    qw = _sc_gather(embedT, idx8.reshape(NROWS))
    res = _matmul(x, qw)
    flag = jnp.asarray(use_qw) != 0
    diff = jnp.where(flag, diff11.reshape(()), jnp.float32(0.0))
    return (res, diff)
